# Initial kernel scaffold; baseline (speedup 1.0000x reference)
import jax
import jax.numpy as jnp
from jax.experimental import pallas as pl


def kernel(input_pool, pool_mask):
    raise NotImplementedError("write your pallas kernel here")


# speedup vs baseline: 13.8770x; 13.8770x over previous
import functools

import jax
import jax.numpy as jnp
from jax import lax
from jax.experimental import pallas as pl
from jax.experimental.pallas import tpu as pltpu
from jax.experimental.pallas import tpu_sc as plsc

_B = 2
_H = 256
_W = 256
_C = 96
_HW = _H * _W
_HOWO = 4 * _HW
_NCOL = _B * _C
_NWORK = 32
_COLS_PER_W = _NCOL // _NWORK
_CHUNK = 65536
_NCHUNK = _HOWO // _CHUNK
_WIN = 8192
_NWIN = _HW // _WIN
_VPW = _WIN // 16
_INV3 = 0.3333333432674408


def _sc_body(val_hbm, mask_hbm, out_hbm, idx_v, val_v, acc):
    wid = lax.axis_index("s") * 2 + lax.axis_index("c")
    col0 = wid * _COLS_PER_W

    def task(j, _):
        col = col0 + (j >> 2)
        chunk = j & 3
        src_base = col * _HW
        dst_base = col * _HOWO + chunk * _CHUNK

        def zero(k, _):
            acc[pl.ds(k * 16, 16)] = jnp.zeros((16,), jnp.float32)
            return 0

        lax.fori_loop(0, _CHUNK // 16, zero, 0)

        def window(w, _):
            base = src_base + w * _WIN
            pltpu.sync_copy(mask_hbm.at[pl.ds(base, _WIN)], idx_v)
            pltpu.sync_copy(val_hbm.at[pl.ds(base, _WIN)], val_v)

            def vec(r, _):
                m = idx_v[pl.ds(r * 16, 16)]
                v = val_v[pl.ds(r * 16, 16)]
                x = lax.shift_right_logical(m, 5)
                t = (x.astype(jnp.float32) * _INV3).astype(jnp.int32)
                ok = lax.shift_right_logical(t, 16) == chunk
                off = lax.bitwise_and(t, 0xFFFF)
                plsc.addupdate_scatter(acc, [off], v, mask=ok)
                return 0

            lax.fori_loop(0, _VPW, vec, 0)
            return 0

        lax.fori_loop(0, _NWIN, window, 0)
        pltpu.sync_copy(acc, out_hbm.at[pl.ds(dst_base, _CHUNK)])
        return 0

    lax.fori_loop(0, _COLS_PER_W * _NCHUNK, task, 0)


@jax.jit
def kernel(input_pool, pool_mask):
    B, H, W, C = input_pool.shape
    Ho, Wo = 2 * H, 2 * W
    vals_t = input_pool.reshape(B, H * W, C).transpose(0, 2, 1).reshape(-1)
    mask_t = pool_mask.reshape(B, H * W, C).transpose(0, 2, 1).reshape(-1)

    mesh = plsc.VectorSubcoreMesh(core_axis_name="c", subcore_axis_name="s")
    out_t = pl.kernel(
        _sc_body,
        out_type=jax.ShapeDtypeStruct((_NCOL * _HOWO,), jnp.float32),
        mesh=mesh,
        scratch_types=[
            pltpu.VMEM((_WIN,), jnp.int32),
            pltpu.VMEM((_WIN,), jnp.float32),
            pltpu.VMEM((_CHUNK,), jnp.float32),
        ],
        compiler_params=pltpu.CompilerParams(needs_layout_passes=False),
    )(vals_t, mask_t)

    return (
        out_t.reshape(B, C, Ho * Wo)
        .transpose(0, 2, 1)
        .reshape(B, Ho, Wo, C)
    )


# speedup vs baseline: 23.7345x; 1.7104x over previous
import functools

import jax
import jax.numpy as jnp
from jax import lax
from jax.experimental import pallas as pl
from jax.experimental.pallas import tpu as pltpu
from jax.experimental.pallas import tpu_sc as plsc

_B = 2
_H = 256
_W = 256
_C = 96
_HW = _H * _W
_HOWO = 4 * _HW
_NCOL = _B * _C
_NWORK = 32
_COLS_PER_W = _NCOL // _NWORK
_A = 87424
_LAST = _HOWO - 2 * _A
_NCHUNK = 3
_WIN = 8192
_NWIN = _HW // _WIN
_U = 8
_INV3 = 0.3333333432674408


def _sc_body(val_hbm, mask_hbm, out_hbm,
             idx0, val0, idx1, val1, acc, sem0, sem1):
    wid = lax.axis_index("s") * 2 + lax.axis_index("c")
    col0 = wid * _COLS_PER_W
    bufs = ((idx0, val0), (idx1, val1))
    sems = (sem0, sem1)

    def start_window(src_base, w, parity):
        base = src_base + w * _WIN
        bi, bv = bufs[parity]
        hi = pltpu.async_copy(mask_hbm.at[pl.ds(base, _WIN)], bi, sems[parity])
        hv = pltpu.async_copy(val_hbm.at[pl.ds(base, _WIN)], bv, sems[parity])
        return hi, hv

    def column(i, _):
        col = col0 + i
        src_base = col * _HW

        def chunk_body(ch, _):
            lo = ch * _A

            handles = [None, None]
            handles[0] = start_window(src_base, 0, 0)

            def zero(k, _):
                zbase = k * (16 * _U)
                for u in range(_U):
                    acc[pl.ds(zbase + u * 16, 16)] = jnp.zeros(
                        (16,), jnp.float32)
                return 0

            lax.fori_loop(0, _A // (16 * _U), zero, 0)

            for w in range(_NWIN):
                cur = w % 2
                if w + 1 < _NWIN:
                    handles[(w + 1) % 2] = start_window(
                        src_base, w + 1, (w + 1) % 2)
                hi, hv = handles[cur]
                hi.wait()
                hv.wait()
                bi, bv = bufs[cur]

                def vec_block(r, _, bi=bi, bv=bv):
                    vbase = r * (16 * _U)
                    for u in range(_U):
                        m = bi[pl.ds(vbase + u * 16, 16)]
                        v = bv[pl.ds(vbase + u * 16, 16)]
                        x = lax.shift_right_logical(m, 5)
                        t = (x.astype(jnp.float32) * _INV3).astype(jnp.int32)
                        off = t - lo
                        ok = off.astype(jnp.uint32) < _A
                        offc = jnp.where(ok, off, 0)
                        plsc.addupdate_scatter(acc, [offc], v, mask=ok)
                    return 0

                lax.fori_loop(0, _WIN // (16 * _U), vec_block, 0)

            dst = col * _HOWO + ch * _A

            @pl.when(ch < 2)
            def _():
                pltpu.sync_copy(acc.at[pl.ds(0, _A)],
                                out_hbm.at[pl.ds(dst, _A)])

            @pl.when(ch == 2)
            def _():
                pltpu.sync_copy(acc.at[pl.ds(0, _LAST)],
                                out_hbm.at[pl.ds(dst, _LAST)])

            return 0

        lax.fori_loop(0, _NCHUNK, chunk_body, 0)
        return 0

    lax.fori_loop(0, _COLS_PER_W, column, 0)


@jax.jit
def kernel(input_pool, pool_mask):
    B, H, W, C = input_pool.shape
    Ho, Wo = 2 * H, 2 * W
    vals_t = input_pool.reshape(B, H * W, C).transpose(0, 2, 1).reshape(-1)
    mask_t = pool_mask.reshape(B, H * W, C).transpose(0, 2, 1).reshape(-1)

    mesh = plsc.VectorSubcoreMesh(core_axis_name="c", subcore_axis_name="s")
    out_t = pl.kernel(
        _sc_body,
        out_type=jax.ShapeDtypeStruct((_NCOL * _HOWO,), jnp.float32),
        mesh=mesh,
        scratch_types=[
            pltpu.VMEM((_WIN,), jnp.int32),
            pltpu.VMEM((_WIN,), jnp.float32),
            pltpu.VMEM((_WIN,), jnp.int32),
            pltpu.VMEM((_WIN,), jnp.float32),
            pltpu.VMEM((_A,), jnp.float32),
            pltpu.SemaphoreType.DMA,
            pltpu.SemaphoreType.DMA,
        ],
        compiler_params=pltpu.CompilerParams(needs_layout_passes=False),
    )(vals_t, mask_t)

    return (
        out_t.reshape(B, C, Ho * Wo)
        .transpose(0, 2, 1)
        .reshape(B, Ho, Wo, C)
    )
